# baseline (speedup 1.0000x reference)
import functools

import jax
import jax.numpy as jnp
from jax import lax
from jax.experimental import pallas as pl
from jax.experimental.pallas import tpu as pltpu
from jax.experimental.pallas import tpu_sc as plsc

N = 10000
E = 320000
D = 128
NC = 2
NS = 16
CH = 128
NCHUNK = E // CH
CPC = NCHUNK // NC
N_PAD = 10112
RPS = N_PAD // NS


def _mm_body(f_ref, w_ref, o_ref):
    o_ref[...] = jnp.dot(f_ref[...], w_ref[...],
                         preferred_element_type=jnp.float32)


def _matmul(features, weight):
    BN = 1000
    return pl.pallas_call(
        _mm_body,
        grid=(N // BN,),
        in_specs=[
            pl.BlockSpec((BN, D), lambda i: (i, 0)),
            pl.BlockSpec((D, D), lambda i: (0, 0)),
        ],
        out_specs=pl.BlockSpec((BN, D), lambda i: (i, 0)),
        out_shape=jax.ShapeDtypeStruct((N, D), jnp.float32),
    )(features, weight)


def _add_body(a_ref, b_ref, o_ref):
    o_ref[...] = a_ref[0] + b_ref[0]


def _combine(parts):
    BN = 1000
    return pl.pallas_call(
        _add_body,
        grid=(N // BN,),
        in_specs=[
            pl.BlockSpec((1, BN, D), lambda i: (0, i, 0)),
            pl.BlockSpec((1, BN, D), lambda i: (1, i, 0)),
        ],
        out_specs=pl.BlockSpec((BN, D), lambda i: (i, 0)),
        out_shape=jax.ShapeDtypeStruct((N, D), jnp.float32),
    )(parts, parts)


def _spmm_body(sup_hbm, ridx_hbm, adj_hbm, out_hbm, accum,
               ibuf0, adjv0, rows0, srow0, isem0, gsem0, ssem0,
               ibuf1, adjv1, rows1, srow1, isem1, gsem1, ssem1,
               ibuf2, adjv2, rows2, srow2, isem2, gsem2, ssem2):
    c = lax.axis_index("c")
    s = lax.axis_index("s")
    IB = (ibuf0, ibuf1, ibuf2)
    AV = (adjv0, adjv1, adjv2)
    RW = (rows0, rows1, rows2)
    SR = (srow0, srow1, srow2)
    ISEM = (isem0, isem1, isem2)
    GSEM = (gsem0, gsem1, gsem2)
    SSEM = (ssem0, ssem1, ssem2)

    zero16 = jnp.zeros((16,), jnp.float32)

    def zb_body(i, _):
        for j in range(D // 16):
            rows0[i, pl.ds(j * 16, 16)] = zero16
        return 0

    lax.fori_loop(0, CH, zb_body, 0)
    r0 = s * RPS

    def zfill(n):
        for k in range(n // 128):
            pltpu.sync_copy(rows0, accum.at[pl.ds(r0 + k * 128, 128)])
        if n % 128:
            pltpu.sync_copy(rows0.at[pl.ds(0, n % 128)],
                            accum.at[pl.ds(r0 + (n // 128) * 128, n % 128)])

    @pl.when(s < NS - 1)
    def _zero_main():
        zfill(RPS)

    @pl.when(s == NS - 1)
    def _zero_tail():
        zfill(N - (NS - 1) * RPS)

    plsc.subcore_barrier()

    nk = CPC // NS + jnp.where(s < CPC % NS, 1, 0)

    def ecopies(li, b):
        base = (c * CPC + s + li * NS) * CH
        return (pltpu.make_async_copy(
                    ridx_hbm.at[pl.ds(0, 2), pl.ds(base, CH)],
                    IB[b], ISEM[b]),
                pltpu.make_async_copy(
                    adj_hbm.at[pl.ds(base, CH)], AV[b], ISEM[b]))

    def idx_issue(li, b):
        @pl.when(li < nk)
        def _():
            for cp in ecopies(li, b):
                cp.start()

    def idx_wait(li, b):
        @pl.when(li < nk)
        def _():
            for cp in ecopies(li, b):
                cp.wait()

    HH = CH // 2

    def gather_issue(li, b):
        @pl.when(li < nk)
        def _():
            pltpu.async_copy(sup_hbm.at[IB[b].at[1, pl.ds(0, HH)]],
                             RW[b].at[pl.ds(0, HH)], GSEM[b])
            pltpu.async_copy(sup_hbm.at[IB[b].at[1, pl.ds(HH, HH)]],
                             RW[b].at[pl.ds(HH, HH)], GSEM[b])

    def gather_wait(li, b):
        @pl.when(li < nk)
        def _():
            pltpu.make_async_copy(sup_hbm.at[IB[b].at[1, pl.ds(0, HH)]],
                                  RW[b].at[pl.ds(0, HH)], GSEM[b]).wait()
            pltpu.make_async_copy(sup_hbm.at[IB[b].at[1, pl.ds(HH, HH)]],
                                  RW[b].at[pl.ds(HH, HH)], GSEM[b]).wait()

    def compute(li, b):
        @pl.when(li < nk)
        def _():
            ib, av, rw, sr = IB[b], AV[b], RW[b], SR[b]

            def mul_body(g, _):
                a16 = av[pl.ds(g * 16, 16)]
                for l in range(16):
                    a = a16[l]
                    e = g * 16 + l
                    for j in range(D // 16):
                        rw[e, pl.ds(j * 16, 16)] = (
                            rw[e, pl.ds(j * 16, 16)] * a)
                return 0

            lax.fori_loop(0, CH // 16, mul_body, 0, unroll=2)
            for j in range(CH // 16):
                sr[pl.ds(j * 16, 16)] = ib[0, pl.ds(j * 16, 16)]

    def scatter_issue(li, b):
        @pl.when(li < nk)
        def _():
            pltpu.async_copy(RW[b], accum.at[SR[b]], SSEM[b], add=True)

    def scatter_wait(li, b):
        @pl.when(jnp.logical_and(li >= 0, li < nk))
        def _():
            pltpu.make_async_copy(RW[b], accum.at[SR[b]], SSEM[b]).wait()

    def slot(li, b):
        bn = (b + 1) % 3
        bp = (b + 2) % 3
        scatter_wait(li - 2, bn)
        idx_wait(li + 1, bn)
        gather_issue(li + 1, bn)
        idx_issue(li + 2, bp)
        gather_wait(li, b)
        compute(li, b)
        scatter_issue(li, b)

    idx_issue(jnp.int32(0), 0)
    idx_issue(jnp.int32(1), 1)
    idx_wait(jnp.int32(0), 0)
    gather_issue(jnp.int32(0), 0)

    def tri_body(i3, _):
        for u in range(3):
            slot(i3 * 3 + u, u)
        return 0

    lax.fori_loop(0, (CPC // NS + 3) // 3 + 1, tri_body, 0)
    plsc.subcore_barrier()

    @pl.when(s < NS - 1)
    def _copy_main():
        pltpu.sync_copy(accum.at[pl.ds(r0, RPS)],
                        out_hbm.at[c, pl.ds(r0, RPS)])

    @pl.when(s == NS - 1)
    def _copy_tail():
        pltpu.sync_copy(accum.at[pl.ds((NS - 1) * RPS, N - (NS - 1) * RPS)],
                        out_hbm.at[c, pl.ds((NS - 1) * RPS,
                                            N - (NS - 1) * RPS)])


def _buf_set():
    return [
        pltpu.VMEM((2, CH), jnp.int32),
        pltpu.VMEM((CH,), jnp.float32),
        pltpu.VMEM((CH, D), jnp.float32),
        pltpu.VMEM((CH,), jnp.int32),
        pltpu.SemaphoreType.DMA,
        pltpu.SemaphoreType.DMA,
        pltpu.SemaphoreType.DMA,
    ]


_spmm = functools.partial(
    pl.kernel,
    out_type=jax.ShapeDtypeStruct((NC, N, D), jnp.float32),
    mesh=plsc.VectorSubcoreMesh(core_axis_name="c", subcore_axis_name="s"),
    scratch_types=[pltpu.VMEM_SHARED((N, D), jnp.float32)]
    + _buf_set() + _buf_set() + _buf_set(),
)(_spmm_body)


def kernel(features, edge_index, adj_vals, weight):
    support = _matmul(features, weight)
    ridx = edge_index.astype(jnp.int32).reshape(2, E)
    parts = _spmm(support, ridx, adj_vals)
    return _combine(parts)


# speedup vs baseline: 1.0144x; 1.0144x over previous
import functools

import jax
import jax.numpy as jnp
from jax import lax
from jax.experimental import pallas as pl
from jax.experimental.pallas import tpu as pltpu
from jax.experimental.pallas import tpu_sc as plsc

N = 10000
E = 320000
D = 128
NC = 2
NS = 16
CH = 128
NCHUNK = E // CH
CPC = NCHUNK // NC
N_PAD = 10112
RPS = N_PAD // NS


def _mm_body(f_ref, w_ref, o_ref):
    o_ref[...] = jnp.dot(f_ref[...], w_ref[...],
                         preferred_element_type=jnp.float32)


def _matmul(features, weight):
    BN = 1000
    return pl.pallas_call(
        _mm_body,
        grid=(N // BN,),
        in_specs=[
            pl.BlockSpec((BN, D), lambda i: (i, 0)),
            pl.BlockSpec((D, D), lambda i: (0, 0)),
        ],
        out_specs=pl.BlockSpec((BN, D), lambda i: (i, 0)),
        out_shape=jax.ShapeDtypeStruct((N, D), jnp.float32),
    )(features, weight)


def _add_body(a_ref, b_ref, o_ref):
    o_ref[...] = a_ref[0] + b_ref[0]


def _combine(parts):
    BN = 2000
    return pl.pallas_call(
        _add_body,
        grid=(N // BN,),
        in_specs=[
            pl.BlockSpec((1, BN, D), lambda i: (0, i, 0)),
            pl.BlockSpec((1, BN, D), lambda i: (1, i, 0)),
        ],
        out_specs=pl.BlockSpec((BN, D), lambda i: (i, 0)),
        out_shape=jax.ShapeDtypeStruct((N, D), jnp.float32),
    )(parts, parts)


def _spmm_body(sup_hbm, ridx_hbm, adj_hbm, out_hbm, accum,
               ibuf0, adjv0, rows0, srow0, isem0, gsem0, ssem0,
               ibuf1, adjv1, rows1, srow1, isem1, gsem1, ssem1,
               ibuf2, adjv2, rows2, srow2, isem2, gsem2, ssem2):
    c = lax.axis_index("c")
    s = lax.axis_index("s")
    IB = (ibuf0, ibuf1, ibuf2)
    AV = (adjv0, adjv1, adjv2)
    RW = (rows0, rows1, rows2)
    SR = (srow0, srow1, srow2)
    ISEM = (isem0, isem1, isem2)
    GSEM = (gsem0, gsem1, gsem2)
    SSEM = (ssem0, ssem1, ssem2)

    zero16 = jnp.zeros((16,), jnp.float32)

    def zb_body(i, _):
        for j in range(D // 16):
            rows0[i, pl.ds(j * 16, 16)] = zero16
        return 0

    lax.fori_loop(0, CH, zb_body, 0)
    r0 = s * RPS

    def zfill(n):
        for k in range(n // 128):
            pltpu.sync_copy(rows0, accum.at[pl.ds(r0 + k * 128, 128)])
        if n % 128:
            pltpu.sync_copy(rows0.at[pl.ds(0, n % 128)],
                            accum.at[pl.ds(r0 + (n // 128) * 128, n % 128)])

    @pl.when(s < NS - 1)
    def _zero_main():
        zfill(RPS)

    @pl.when(s == NS - 1)
    def _zero_tail():
        zfill(N - (NS - 1) * RPS)

    plsc.subcore_barrier()

    nk = CPC // NS + jnp.where(s < CPC % NS, 1, 0)

    def ecopies(li, b):
        base = (c * CPC + s + li * NS) * CH
        return (pltpu.make_async_copy(
                    ridx_hbm.at[pl.ds(0, 2), pl.ds(base, CH)],
                    IB[b], ISEM[b]),
                pltpu.make_async_copy(
                    adj_hbm.at[pl.ds(base, CH)], AV[b], ISEM[b]))

    def idx_issue(li, b):
        @pl.when(li < nk)
        def _():
            for cp in ecopies(li, b):
                cp.start()

    def idx_wait(li, b):
        @pl.when(li < nk)
        def _():
            for cp in ecopies(li, b):
                cp.wait()

    HH = CH // 2

    def gather_issue(li, b):
        @pl.when(li < nk)
        def _():
            pltpu.async_copy(sup_hbm.at[IB[b].at[1, pl.ds(0, HH)]],
                             RW[b].at[pl.ds(0, HH)], GSEM[b])
            pltpu.async_copy(sup_hbm.at[IB[b].at[1, pl.ds(HH, HH)]],
                             RW[b].at[pl.ds(HH, HH)], GSEM[b])

    def gather_wait(li, b):
        @pl.when(li < nk)
        def _():
            pltpu.make_async_copy(sup_hbm.at[IB[b].at[1, pl.ds(0, HH)]],
                                  RW[b].at[pl.ds(0, HH)], GSEM[b]).wait()
            pltpu.make_async_copy(sup_hbm.at[IB[b].at[1, pl.ds(HH, HH)]],
                                  RW[b].at[pl.ds(HH, HH)], GSEM[b]).wait()

    def compute(li, b):
        @pl.when(li < nk)
        def _():
            ib, av, rw, sr = IB[b], AV[b], RW[b], SR[b]

            def mul_body(g, _):
                a16 = av[pl.ds(g * 16, 16)]
                for l in range(16):
                    a = a16[l]
                    e = g * 16 + l
                    for j in range(D // 16):
                        rw[e, pl.ds(j * 16, 16)] = (
                            rw[e, pl.ds(j * 16, 16)] * a)
                return 0

            lax.fori_loop(0, CH // 16, mul_body, 0, unroll=2)
            for j in range(CH // 16):
                sr[pl.ds(j * 16, 16)] = ib[0, pl.ds(j * 16, 16)]

    def scatter_issue(li, b):
        @pl.when(li < nk)
        def _():
            pltpu.async_copy(RW[b], accum.at[SR[b]], SSEM[b], add=True)

    def scatter_wait(li, b):
        @pl.when(jnp.logical_and(li >= 0, li < nk))
        def _():
            pltpu.make_async_copy(RW[b], accum.at[SR[b]], SSEM[b]).wait()

    def slot(li, b):
        bn = (b + 1) % 3
        bp = (b + 2) % 3
        scatter_wait(li - 2, bn)
        idx_wait(li + 1, bn)
        gather_issue(li + 1, bn)
        idx_issue(li + 2, bp)
        gather_wait(li, b)
        compute(li, b)
        scatter_issue(li, b)

    idx_issue(jnp.int32(0), 0)
    idx_issue(jnp.int32(1), 1)
    idx_wait(jnp.int32(0), 0)
    gather_issue(jnp.int32(0), 0)

    def tri_body(i3, _):
        for u in range(3):
            slot(i3 * 3 + u, u)
        return 0

    lax.fori_loop(0, (CPC // NS + 3) // 3 + 1, tri_body, 0)
    plsc.subcore_barrier()

    @pl.when(s < NS - 1)
    def _copy_main():
        pltpu.sync_copy(accum.at[pl.ds(r0, RPS)],
                        out_hbm.at[c, pl.ds(r0, RPS)])

    @pl.when(s == NS - 1)
    def _copy_tail():
        pltpu.sync_copy(accum.at[pl.ds((NS - 1) * RPS, N - (NS - 1) * RPS)],
                        out_hbm.at[c, pl.ds((NS - 1) * RPS,
                                            N - (NS - 1) * RPS)])


def _buf_set():
    return [
        pltpu.VMEM((2, CH), jnp.int32),
        pltpu.VMEM((CH,), jnp.float32),
        pltpu.VMEM((CH, D), jnp.float32),
        pltpu.VMEM((CH,), jnp.int32),
        pltpu.SemaphoreType.DMA,
        pltpu.SemaphoreType.DMA,
        pltpu.SemaphoreType.DMA,
    ]


_spmm = functools.partial(
    pl.kernel,
    out_type=jax.ShapeDtypeStruct((NC, N, D), jnp.float32),
    mesh=plsc.VectorSubcoreMesh(core_axis_name="c", subcore_axis_name="s"),
    scratch_types=[pltpu.VMEM_SHARED((N, D), jnp.float32)]
    + _buf_set() + _buf_set() + _buf_set(),
)(_spmm_body)


def kernel(features, edge_index, adj_vals, weight):
    support = _matmul(features, weight)
    ridx = edge_index.astype(jnp.int32).reshape(2, E)
    parts = _spmm(support, ridx, adj_vals)
    return _combine(parts)


# speedup vs baseline: 1.0314x; 1.0168x over previous
import functools

import jax
import jax.numpy as jnp
from jax import lax
from jax.experimental import pallas as pl
from jax.experimental.pallas import tpu as pltpu
from jax.experimental.pallas import tpu_sc as plsc

N = 10000
E = 320000
D = 128
NC = 2
NS = 16
CH = 128
NCHUNK = E // CH
CPC = NCHUNK // NC
N_PAD = 10112
RPS = N_PAD // NS


def _mm_body(f_ref, w_ref, o_ref):
    o_ref[...] = jnp.dot(f_ref[...], w_ref[...],
                         preferred_element_type=jnp.float32)


def _matmul(features, weight):
    BN = 2000
    return pl.pallas_call(
        _mm_body,
        grid=(N // BN,),
        in_specs=[
            pl.BlockSpec((BN, D), lambda i: (i, 0)),
            pl.BlockSpec((D, D), lambda i: (0, 0)),
        ],
        out_specs=pl.BlockSpec((BN, D), lambda i: (i, 0)),
        out_shape=jax.ShapeDtypeStruct((N, D), jnp.float32),
    )(features, weight)


def _add_body(a_ref, b_ref, o_ref):
    o_ref[...] = a_ref[0] + b_ref[0]


def _combine(parts):
    BN = 2000
    return pl.pallas_call(
        _add_body,
        grid=(N // BN,),
        in_specs=[
            pl.BlockSpec((1, BN, D), lambda i: (0, i, 0)),
            pl.BlockSpec((1, BN, D), lambda i: (1, i, 0)),
        ],
        out_specs=pl.BlockSpec((BN, D), lambda i: (i, 0)),
        out_shape=jax.ShapeDtypeStruct((N, D), jnp.float32),
    )(parts, parts)


def _spmm_body(sup_hbm, ridx_hbm, adj_hbm, out_hbm, accum,
               ibuf0, adjv0, rows0, srow0, isem0, gsem0, ssem0,
               ibuf1, adjv1, rows1, srow1, isem1, gsem1, ssem1,
               ibuf2, adjv2, rows2, srow2, isem2, gsem2, ssem2):
    c = lax.axis_index("c")
    s = lax.axis_index("s")
    IB = (ibuf0, ibuf1, ibuf2)
    AV = (adjv0, adjv1, adjv2)
    RW = (rows0, rows1, rows2)
    SR = (srow0, srow1, srow2)
    ISEM = (isem0, isem1, isem2)
    GSEM = (gsem0, gsem1, gsem2)
    SSEM = (ssem0, ssem1, ssem2)

    zero16 = jnp.zeros((16,), jnp.float32)

    def zb_body(i, _):
        for j in range(D // 16):
            rows0[i, pl.ds(j * 16, 16)] = zero16
        return 0

    lax.fori_loop(0, CH, zb_body, 0)
    r0 = s * RPS

    def zfill(n):
        for k in range(n // 128):
            pltpu.sync_copy(rows0, accum.at[pl.ds(r0 + k * 128, 128)])
        if n % 128:
            pltpu.sync_copy(rows0.at[pl.ds(0, n % 128)],
                            accum.at[pl.ds(r0 + (n // 128) * 128, n % 128)])

    @pl.when(s < NS - 1)
    def _zero_main():
        zfill(RPS)

    @pl.when(s == NS - 1)
    def _zero_tail():
        zfill(N - (NS - 1) * RPS)

    plsc.subcore_barrier()

    nk = CPC // NS + jnp.where(s < CPC % NS, 1, 0)

    def ecopies(li, b):
        base = (c * CPC + s + li * NS) * CH
        return (pltpu.make_async_copy(
                    ridx_hbm.at[pl.ds(0, 2), pl.ds(base, CH)],
                    IB[b], ISEM[b]),
                pltpu.make_async_copy(
                    adj_hbm.at[pl.ds(base, CH)], AV[b], ISEM[b]))

    def idx_issue(li, b):
        @pl.when(li < nk)
        def _():
            for cp in ecopies(li, b):
                cp.start()

    def idx_wait(li, b):
        @pl.when(li < nk)
        def _():
            for cp in ecopies(li, b):
                cp.wait()

    HH = CH // 2

    def gather_issue(li, b):
        @pl.when(li < nk)
        def _():
            pltpu.async_copy(sup_hbm.at[IB[b].at[1, pl.ds(0, HH)]],
                             RW[b].at[pl.ds(0, HH)], GSEM[b])
            pltpu.async_copy(sup_hbm.at[IB[b].at[1, pl.ds(HH, HH)]],
                             RW[b].at[pl.ds(HH, HH)], GSEM[b])

    def gather_wait(li, b):
        @pl.when(li < nk)
        def _():
            pltpu.make_async_copy(sup_hbm.at[IB[b].at[1, pl.ds(0, HH)]],
                                  RW[b].at[pl.ds(0, HH)], GSEM[b]).wait()
            pltpu.make_async_copy(sup_hbm.at[IB[b].at[1, pl.ds(HH, HH)]],
                                  RW[b].at[pl.ds(HH, HH)], GSEM[b]).wait()

    def compute(li, b):
        @pl.when(li < nk)
        def _():
            ib, av, rw, sr = IB[b], AV[b], RW[b], SR[b]

            def mul_body(g, _):
                a16 = av[pl.ds(g * 16, 16)]
                for l in range(16):
                    a = a16[l]
                    e = g * 16 + l
                    for j in range(D // 16):
                        rw[e, pl.ds(j * 16, 16)] = (
                            rw[e, pl.ds(j * 16, 16)] * a)
                return 0

            lax.fori_loop(0, CH // 16, mul_body, 0, unroll=2)
            for j in range(CH // 16):
                sr[pl.ds(j * 16, 16)] = ib[0, pl.ds(j * 16, 16)]

    def scatter_issue(li, b):
        @pl.when(li < nk)
        def _():
            pltpu.async_copy(RW[b], accum.at[SR[b]], SSEM[b], add=True)

    def scatter_wait(li, b):
        @pl.when(jnp.logical_and(li >= 0, li < nk))
        def _():
            pltpu.make_async_copy(RW[b], accum.at[SR[b]], SSEM[b]).wait()

    def slot(li, b):
        bn = (b + 1) % 3
        bp = (b + 2) % 3
        scatter_wait(li - 2, bn)
        idx_wait(li + 1, bn)
        gather_issue(li + 1, bn)
        idx_issue(li + 2, bp)
        gather_wait(li, b)
        compute(li, b)
        scatter_issue(li, b)

    idx_issue(jnp.int32(0), 0)
    idx_issue(jnp.int32(1), 1)
    idx_wait(jnp.int32(0), 0)
    gather_issue(jnp.int32(0), 0)

    def tri_body(i3, _):
        for u in range(3):
            slot(i3 * 3 + u, u)
        return 0

    lax.fori_loop(0, (CPC // NS + 3) // 3 + 1, tri_body, 0)
    plsc.subcore_barrier()

    @pl.when(s < NS - 1)
    def _copy_main():
        pltpu.sync_copy(accum.at[pl.ds(r0, RPS)],
                        out_hbm.at[c, pl.ds(r0, RPS)])

    @pl.when(s == NS - 1)
    def _copy_tail():
        pltpu.sync_copy(accum.at[pl.ds((NS - 1) * RPS, N - (NS - 1) * RPS)],
                        out_hbm.at[c, pl.ds((NS - 1) * RPS,
                                            N - (NS - 1) * RPS)])


def _buf_set():
    return [
        pltpu.VMEM((2, CH), jnp.int32),
        pltpu.VMEM((CH,), jnp.float32),
        pltpu.VMEM((CH, D), jnp.float32),
        pltpu.VMEM((CH,), jnp.int32),
        pltpu.SemaphoreType.DMA,
        pltpu.SemaphoreType.DMA,
        pltpu.SemaphoreType.DMA,
    ]


_spmm = functools.partial(
    pl.kernel,
    out_type=jax.ShapeDtypeStruct((NC, N, D), jnp.float32),
    mesh=plsc.VectorSubcoreMesh(core_axis_name="c", subcore_axis_name="s"),
    scratch_types=[pltpu.VMEM_SHARED((N, D), jnp.float32)]
    + _buf_set() + _buf_set() + _buf_set(),
)(_spmm_body)


def kernel(features, edge_index, adj_vals, weight):
    support = _matmul(features, weight)
    ridx = edge_index.astype(jnp.int32).reshape(2, E)
    parts = _spmm(support, ridx, adj_vals)
    return _combine(parts)
